# baseline (speedup 1.0000x reference)
import functools

import jax
import jax.numpy as jnp
from jax import lax
from jax.experimental import pallas as pl
from jax.experimental.pallas import tpu as pltpu
from jax.experimental.pallas import tpu_sc as plsc

_VOCAB = 1000000
_HIDDEN = 64
_BATCH = 16384
_N_FIELDS = 26

_ROWS = _BATCH * _N_FIELDS
_NC = 2
_NS = 16
_NW = _NC * _NS
_R_PER_W = _ROWS // _NW
_BLK = 128
_NBLK = _R_PER_W // _BLK

_mesh = plsc.VectorSubcoreMesh(core_axis_name="c", subcore_axis_name="s")


@functools.partial(
    pl.kernel,
    out_type=jax.ShapeDtypeStruct((_ROWS, _HIDDEN), jnp.float32),
    mesh=_mesh,
    compiler_params=pltpu.CompilerParams(use_tc_tiling_on_sc=False),
    scratch_types=[
        pltpu.VMEM((_NBLK, _BLK), jnp.int32),
        pltpu.VMEM((_BLK, _HIDDEN), jnp.float32),
        pltpu.SemaphoreType.DMA,
    ],
)
def _gather_kernel(idx_hbm, table_hbm, out_hbm, idx_v, rows_v, sem):
    wid = lax.axis_index("s") * _NC + lax.axis_index("c")
    base_blk = wid * _NBLK
    pltpu.sync_copy(idx_hbm.at[pl.ds(base_blk, _NBLK)], idx_v)

    def body(g, _):
        pltpu.async_copy(table_hbm.at[idx_v.at[g]], rows_v, sem).wait()
        pltpu.sync_copy(rows_v, out_hbm.at[pl.ds((base_blk + g) * _BLK, _BLK)])
        return ()

    lax.fori_loop(0, _NBLK, body, (), unroll=False)


def kernel(x, table):
    idx = x.reshape(_ROWS // _BLK, _BLK)
    out = _gather_kernel(idx, table)
    return out.reshape(_BATCH, _N_FIELDS * _HIDDEN)
